# baseline (speedup 1.0000x reference)
import functools

import jax
import jax.numpy as jnp
from jax import lax
from jax.experimental import pallas as pl
from jax.experimental.pallas import tpu as pltpu
from jax.experimental.pallas import tpu_sc as plsc

_NUM_CORES = 2
_NUM_SUBCORES = 16
_NUM_WORKERS = _NUM_CORES * _NUM_SUBCORES
_CHUNK = 1280


@functools.partial(jax.jit, static_argnames=("b_per_w", "n_chunks", "d"))
def _sc_lookup(x_flat, table, *, b_per_w, n_chunks, d):
    mesh = plsc.VectorSubcoreMesh(
        core_axis_name="c", subcore_axis_name="s",
        num_cores=_NUM_CORES, num_subcores=_NUM_SUBCORES)

    @functools.partial(
        pl.kernel,
        out_type=jax.ShapeDtypeStruct((x_flat.shape[0], d), table.dtype),
        mesh=mesh,
        compiler_params=pltpu.CompilerParams(use_tc_tiling_on_sc=False),
        scratch_types=[
            pltpu.VMEM((b_per_w,), jnp.int32),
            pltpu.VMEM((2, _CHUNK, d), table.dtype),
            pltpu.SemaphoreType.DMA,
        ],
    )
    def run(x_hbm, table_hbm, out_hbm, idx_v, rows_v, gsem):
        wid = lax.axis_index("s") * _NUM_CORES + lax.axis_index("c")
        base = pl.multiple_of(wid * b_per_w, b_per_w)
        pltpu.sync_copy(x_hbm.at[pl.ds(base, b_per_w)], idx_v)

        def gather(g, buf):
            off = pl.multiple_of(g * _CHUNK, _CHUNK)
            return pltpu.async_copy(
                table_hbm.at[idx_v.at[pl.ds(off, _CHUNK)]],
                rows_v.at[buf], gsem)

        gather(0, 0)

        def pair(k, carry):
            for par in (0, 1):
                g = 2 * k + par
                @pl.when(g + 1 < n_chunks)
                def _():
                    gather(g + 1, 1 - par)
                pltpu.make_async_copy(
                    table_hbm.at[idx_v.at[pl.ds(0, _CHUNK)]],
                    rows_v.at[par], gsem).wait()
                off = pl.multiple_of(g * _CHUNK, _CHUNK)
                pltpu.sync_copy(rows_v.at[par],
                                out_hbm.at[pl.ds(base + off, _CHUNK)])
            return carry

        lax.fori_loop(0, n_chunks // 2, pair, 0)

        if n_chunks % 2:
            g = n_chunks - 1
            par = g % 2
            pltpu.make_async_copy(
                table_hbm.at[idx_v.at[pl.ds(0, _CHUNK)]],
                rows_v.at[par], gsem).wait()
            off = pl.multiple_of(g * _CHUNK, _CHUNK)
            pltpu.sync_copy(rows_v.at[par],
                            out_hbm.at[pl.ds(base + off, _CHUNK)])

    return run(x_flat, table)


def kernel(x, table):
    b, h = x.shape
    v, d = table.shape
    n = b * h
    assert n % _NUM_WORKERS == 0
    b_per_w = n // _NUM_WORKERS
    assert b_per_w % (2 * _CHUNK) == 0 or b_per_w % _CHUNK == 0
    x_flat = x.reshape(n).astype(jnp.int32)
    out = _sc_lookup(x_flat, table, b_per_w=b_per_w,
                     n_chunks=b_per_w // _CHUNK, d=d)
    return out.reshape(b, h, d)


# speedup vs baseline: 1.0006x; 1.0006x over previous
import functools

import jax
import jax.numpy as jnp
from jax import lax
from jax.experimental import pallas as pl
from jax.experimental.pallas import tpu as pltpu
from jax.experimental.pallas import tpu_sc as plsc

_NUM_CORES = 2
_NUM_SUBCORES = 16
_NUM_WORKERS = _NUM_CORES * _NUM_SUBCORES
_CHUNK = 1024


@functools.partial(jax.jit, static_argnames=("b_per_w", "n_chunks", "d"))
def _sc_lookup(x_flat, table, *, b_per_w, n_chunks, d):
    mesh = plsc.VectorSubcoreMesh(
        core_axis_name="c", subcore_axis_name="s",
        num_cores=_NUM_CORES, num_subcores=_NUM_SUBCORES)

    @functools.partial(
        pl.kernel,
        out_type=jax.ShapeDtypeStruct((x_flat.shape[0], d), table.dtype),
        mesh=mesh,
        compiler_params=pltpu.CompilerParams(use_tc_tiling_on_sc=False),
        scratch_types=[
            pltpu.VMEM((b_per_w,), jnp.int32),
            pltpu.VMEM((2, _CHUNK, d), table.dtype),
            pltpu.SemaphoreType.DMA,
        ],
    )
    def run(x_hbm, table_hbm, out_hbm, idx_v, rows_v, gsem):
        wid = lax.axis_index("s") * _NUM_CORES + lax.axis_index("c")
        base = pl.multiple_of(wid * b_per_w, b_per_w)
        pltpu.sync_copy(x_hbm.at[pl.ds(base, b_per_w)], idx_v)

        def gather(g, buf):
            off = pl.multiple_of(g * _CHUNK, _CHUNK)
            return pltpu.async_copy(
                table_hbm.at[idx_v.at[pl.ds(off, _CHUNK)]],
                rows_v.at[buf], gsem)

        gather(0, 0)

        def pair(k, carry):
            for par in (0, 1):
                g = 2 * k + par
                @pl.when(g + 1 < n_chunks)
                def _():
                    gather(g + 1, 1 - par)
                pltpu.make_async_copy(
                    table_hbm.at[idx_v.at[pl.ds(0, _CHUNK)]],
                    rows_v.at[par], gsem).wait()
                off = pl.multiple_of(g * _CHUNK, _CHUNK)
                pltpu.sync_copy(rows_v.at[par],
                                out_hbm.at[pl.ds(base + off, _CHUNK)])
            return carry

        lax.fori_loop(0, n_chunks // 2, pair, 0)

        if n_chunks % 2:
            g = n_chunks - 1
            par = g % 2
            pltpu.make_async_copy(
                table_hbm.at[idx_v.at[pl.ds(0, _CHUNK)]],
                rows_v.at[par], gsem).wait()
            off = pl.multiple_of(g * _CHUNK, _CHUNK)
            pltpu.sync_copy(rows_v.at[par],
                            out_hbm.at[pl.ds(base + off, _CHUNK)])

    return run(x_flat, table)


def kernel(x, table):
    b, h = x.shape
    v, d = table.shape
    n = b * h
    assert n % _NUM_WORKERS == 0
    b_per_w = n // _NUM_WORKERS
    assert b_per_w % (2 * _CHUNK) == 0 or b_per_w % _CHUNK == 0
    x_flat = x.reshape(n).astype(jnp.int32)
    out = _sc_lookup(x_flat, table, b_per_w=b_per_w,
                     n_chunks=b_per_w // _CHUNK, d=d)
    return out.reshape(b, h, d)
